# Initial kernel scaffold; baseline (speedup 1.0000x reference)
import functools
import math

import jax
import jax.numpy as jnp
from jax import lax
from jax.experimental import pallas as pl
from jax.experimental.pallas import tpu as pltpu
from jax.experimental.pallas import tpu_sc as plsc

VOCAB = 100000
EMBED = 64
MAX_LEN = 512
BATCH = 1024
SEQ = 200

NC, NS, L = 2, 16, 16
NW = NC * NS
B = BATCH * SEQ
B_PER_W = B // NW
CH = 400
NCHUNK = B_PER_W // CH
G = 80
NG = CH // G
NV = EMBED // L


def _positional(max_len, d):
    position = jnp.arange(max_len, dtype=jnp.float32)[:, None]
    div = jnp.exp(
        jnp.arange(0, d, 2, dtype=jnp.float32) * (-math.log(10000.0) / d)
    )
    pe = jnp.zeros((max_len, d), jnp.float32)
    pe = pe.at[:, 0::2].set(jnp.sin(position * div))
    pe = pe.at[:, 1::2].set(jnp.cos(position * div))
    return pe


def _body(seq_g, seq_m, table, pe_rep, out, idx_v, idx_m, rows_v, pe_v, sem):
    wid = lax.axis_index("s") * NC + lax.axis_index("c")
    pltpu.sync_copy(pe_rep, pe_v)

    def chunk(c, _):
        gc = wid * NCHUNK + c
        pltpu.sync_copy(seq_g.at[pl.ds(gc * NG, NG)], idx_v)
        pltpu.sync_copy(seq_m.at[pl.ds(gc * (CH // L), CH // L)], idx_m)
        copies = [
            pltpu.async_copy(
                table.at[idx_v.at[k]], rows_v.at[pl.ds(k * G, G)], sem
            )
            for k in range(NG)
        ]
        for cp in copies:
            cp.wait()

        def fix_group(g, _):
            iv = idx_m[g, :]

            @pl.when(jnp.min(iv) == 0)
            def _slow():
                def fix_row(j, _):
                    s = idx_m[g, j]
                    w = jnp.where(s == 0, 0.0, 1.0).astype(jnp.float32)
                    row = g * L + j
                    for k in range(NV):
                        sl = pl.ds(k * L, L)
                        rows_v[row, sl] = rows_v[row, sl] * w
                    return 0

                lax.fori_loop(0, L, fix_row, 0)

            return 0

        lax.fori_loop(0, CH // L, fix_group, 0)

        @plsc.parallel_loop(0, CH, step=2, unroll=4)
        def _add(j):
            for r in range(2):
                for k in range(NV):
                    sl = pl.ds(k * L, L)
                    rows_v[j + r, sl] = rows_v[j + r, sl] + pe_v[j + r, sl]

        pltpu.sync_copy(rows_v, out.at[pl.ds(gc * CH, CH)])
        return 0

    lax.fori_loop(0, NCHUNK, chunk, 0)


@jax.jit
def kernel(sequence, token_table):
    seq_flat = sequence.reshape(-1).astype(jnp.int32)
    seq_g = seq_flat.reshape(B // G, G)
    seq_m = seq_flat.reshape(B // L, L)
    pe_rep = jnp.tile(_positional(MAX_LEN, EMBED)[:SEQ], (CH // SEQ, 1))

    run = functools.partial(
        pl.kernel,
        out_type=jax.ShapeDtypeStruct((B, EMBED), jnp.float32),
        mesh=plsc.VectorSubcoreMesh(core_axis_name="c", subcore_axis_name="s"),
        scratch_types=[
            pltpu.VMEM((NG, G), jnp.int32),
            pltpu.VMEM((CH // L, L), jnp.int32),
            pltpu.VMEM((CH, EMBED), jnp.float32),
            pltpu.VMEM((CH, EMBED), jnp.float32),
            pltpu.SemaphoreType.DMA,
        ],
    )(_body)
    out = run(seq_g, seq_m, token_table, pe_rep)
    return out.reshape(BATCH, SEQ, EMBED)


# speedup vs baseline: 2.7414x; 2.7414x over previous
import functools
import math

import jax
import jax.numpy as jnp
from jax import lax
from jax.experimental import pallas as pl
from jax.experimental.pallas import tpu as pltpu
from jax.experimental.pallas import tpu_sc as plsc

VOCAB = 100000
EMBED = 64
MAX_LEN = 512
BATCH = 1024
SEQ = 200

NC, NS, L = 2, 16, 16
NW = NC * NS
B = BATCH * SEQ
B_PER_W = B // NW
CH = 400
NCHUNK = B_PER_W // CH
G = 80
NG = CH // G
NV = EMBED // L


def _positional(max_len, d):
    position = jnp.arange(max_len, dtype=jnp.float32)[:, None]
    div = jnp.exp(
        jnp.arange(0, d, 2, dtype=jnp.float32) * (-math.log(10000.0) / d)
    )
    pe = jnp.zeros((max_len, d), jnp.float32)
    pe = pe.at[:, 0::2].set(jnp.sin(position * div))
    pe = pe.at[:, 1::2].set(jnp.cos(position * div))
    return pe


def _body(seq_g, table, pe_rep, out, idx_g, rows_v, pe_v, sem):
    wid = lax.axis_index("s") * NC + lax.axis_index("c")
    pltpu.sync_copy(pe_rep, pe_v)
    pltpu.sync_copy(seq_g.at[pl.ds(wid * (B_PER_W // G), B_PER_W // G)], idx_g)

    def chunk(c, _):
        gc = wid * NCHUNK + c
        copies = [
            pltpu.async_copy(
                table.at[idx_g.at[c * NG + k]], rows_v.at[pl.ds(k * G, G)], sem
            )
            for k in range(NG)
        ]
        for cp in copies:
            cp.wait()

        def fix_group(r, _):
            irow = c * NG + r
            for sub in range(G // L):
                iv = idx_g[irow, pl.ds(sub * L, L)]
                zmask = iv == 0
                nzero = plsc.all_reduce_population_count(zmask)[0]

                @pl.when(nzero > 0)
                def _slow():
                    def fix_row(j, _):
                        rowsel = jnp.full((L,), irow, jnp.int32)
                        colsel = jnp.full((L,), sub * L + j, jnp.int32)
                        s = plsc.load_gather(idx_g, [rowsel, colsel])
                        w = jnp.where(
                            s == 0, jnp.float32(0.0), jnp.float32(1.0)
                        )
                        row = r * G + sub * L + j
                        for k in range(NV):
                            sl = pl.ds(k * L, L)
                            rows_v[row, sl] = rows_v[row, sl] * w
                        return 0

                    lax.fori_loop(0, L, fix_row, 0)

            return 0

        lax.fori_loop(0, NG, fix_group, 0)

        @plsc.parallel_loop(0, CH, step=2, unroll=4)
        def _add(j):
            for r in range(2):
                for k in range(NV):
                    sl = pl.ds(k * L, L)
                    rows_v[j + r, sl] = rows_v[j + r, sl] + pe_v[j + r, sl]

        pltpu.sync_copy(rows_v, out.at[pl.ds(gc * CH, CH)])
        return 0

    lax.fori_loop(0, NCHUNK, chunk, 0)


@jax.jit
def kernel(sequence, token_table):
    seq_flat = sequence.reshape(-1).astype(jnp.int32)
    seq_g = seq_flat.reshape(B // G, G)
    pe_rep = jnp.tile(_positional(MAX_LEN, EMBED)[:SEQ], (CH // SEQ, 1))

    run = functools.partial(
        pl.kernel,
        out_type=jax.ShapeDtypeStruct((B, EMBED), jnp.float32),
        mesh=plsc.VectorSubcoreMesh(core_axis_name="c", subcore_axis_name="s"),
        compiler_params=pltpu.CompilerParams(
            needs_layout_passes=False, use_tc_tiling_on_sc=False
        ),
        scratch_types=[
            pltpu.VMEM((B_PER_W // G, G), jnp.int32),
            pltpu.VMEM((CH, EMBED), jnp.float32),
            pltpu.VMEM((CH, EMBED), jnp.float32),
            pltpu.SemaphoreType.DMA,
        ],
    )(_body)
    out = run(seq_g, token_table, pe_rep)
    return out.reshape(BATCH, SEQ, EMBED)
